# baseline (speedup 1.0000x reference)
import functools

import jax
import jax.numpy as jnp
from jax import lax
from jax.experimental import pallas as pl
from jax.experimental.pallas import tpu as pltpu
from jax.experimental.pallas import tpu_sc as plsc

B, C, D = 4, 4096, 1024
NC, NS = 2, 16
NW = NC * NS
C_PER_W = C // NW
RC = 4
NCHUNK = C_PER_W // RC
NPAIR = NCHUNK // 2
NVU = 16
NVJ = D // (16 * NVU)


def _sc_body(x_hbm, emb_hbm, out_hbm, ebuf, xbuf, obuf, sem_in0, sem_in1,
             sem_out0, sem_out1):
    cid = lax.axis_index("c")
    sid = lax.axis_index("s")
    w = sid * NC + cid
    c0 = w * C_PER_W
    sems_in = (sem_in0, sem_in1)
    sems_out = (sem_out0, sem_out1)

    def load_copies(k, p):
        lo = c0 + k * RC
        return [
            pltpu.make_async_copy(emb_hbm.at[pl.ds(lo, RC)], ebuf.at[p],
                                  sems_in[p]),
            pltpu.make_async_copy(x_hbm.at[:, pl.ds(lo, RC)], xbuf.at[p],
                                  sems_in[p]),
        ]

    def store_copies(k, p):
        lo = c0 + k * RC
        return [pltpu.make_async_copy(
            obuf.at[p], out_hbm.at[:, pl.ds(lo, RC)], sems_out[p])]

    for cp in load_copies(0, 0) + load_copies(1, 1):
        cp.start()

    def pair(kk, carry):
        for p in range(2):
            k = 2 * kk + p
            for cp in load_copies(k, p):
                cp.wait()

            @pl.when(kk >= 1)
            def _drain():
                for cp in store_copies(k - 2, p):
                    cp.wait()

            for r in range(RC):
                def _add(j, c2, r=r):
                    for u in range(NVU):
                        off = (j * NVU + u) * 16
                        e = ebuf[p, r, pl.ds(off, 16)]
                        for b in range(B):
                            obuf[p, b, r, pl.ds(off, 16)] = (
                                xbuf[p, b, r, pl.ds(off, 16)] + e)
                    return c2

                lax.fori_loop(0, NVJ, _add, 0)

            for cp in store_copies(k, p):
                cp.start()

            @pl.when(kk + 1 < NPAIR)
            def _prefetch():
                for cp in load_copies(k + 2, p):
                    cp.start()
        return carry

    lax.fori_loop(0, NPAIR, pair, 0)

    for p in range(2):
        for cp in store_copies(NCHUNK - 2 + p, p):
            cp.wait()


@functools.partial(
    pl.kernel,
    mesh=plsc.VectorSubcoreMesh(core_axis_name="c", subcore_axis_name="s"),
    out_type=jax.ShapeDtypeStruct((B, C, D), jnp.float32),
    scratch_types=[
        pltpu.VMEM((2, RC, D), jnp.float32),
        pltpu.VMEM((2, B, RC, D), jnp.float32),
        pltpu.VMEM((2, B, RC, D), jnp.float32),
        pltpu.SemaphoreType.DMA,
        pltpu.SemaphoreType.DMA,
        pltpu.SemaphoreType.DMA,
        pltpu.SemaphoreType.DMA,
    ],
)
def _sc_kernel(x_hbm, emb_hbm, out_hbm, ebuf, xbuf, obuf, sem_in0, sem_in1,
               sem_out0, sem_out1):
    _sc_body(x_hbm, emb_hbm, out_hbm, ebuf, xbuf, obuf, sem_in0, sem_in1,
             sem_out0, sem_out1)


def kernel(x, embedding):
    return _sc_kernel(x, embedding)


# speedup vs baseline: 1.0656x; 1.0656x over previous
import functools

import jax
import jax.numpy as jnp
from jax import lax
from jax.experimental import pallas as pl
from jax.experimental.pallas import tpu as pltpu
from jax.experimental.pallas import tpu_sc as plsc

B, C, D = 4, 4096, 1024
NC, NS = 2, 16
NW = NC * NS
C_PER_W = C // NW
RC = 2
NCHUNK = C_PER_W // RC
NPAIR = NCHUNK // 2
NVJ = D // (16 * 8)


def _sc_body(x_hbm, emb_hbm, out_hbm, ebuf, xbuf, obuf, sem_in0, sem_in1,
             sem_out0, sem_out1):
    cid = lax.axis_index("c")
    sid = lax.axis_index("s")
    w = sid * NC + cid
    c0 = w * C_PER_W
    sems_in = (sem_in0, sem_in1)
    sems_out = (sem_out0, sem_out1)

    def load_copies(k, p):
        lo = c0 + k * RC
        return [
            pltpu.make_async_copy(emb_hbm.at[pl.ds(lo, RC)], ebuf.at[p],
                                  sems_in[p]),
            pltpu.make_async_copy(x_hbm.at[:, pl.ds(lo, RC)], xbuf.at[p],
                                  sems_in[p]),
        ]

    def store_copies(k, p):
        lo = c0 + k * RC
        return [pltpu.make_async_copy(
            obuf.at[p], out_hbm.at[:, pl.ds(lo, RC)], sems_out[p])]

    for cp in load_copies(0, 0) + load_copies(1, 1):
        cp.start()

    def pair(kk, carry):
        for p in range(2):
            k = 2 * kk + p
            for cp in load_copies(k, p):
                cp.wait()

            @pl.when(kk >= 1)
            def _drain():
                for cp in store_copies(k - 2, p):
                    cp.wait()

            for r in range(RC):
                def _add(j, c2, r=r):
                    for u in range(8):
                        off = (j * 8 + u) * 16
                        e = ebuf[p, r, pl.ds(off, 16)]
                        for b in range(B):
                            obuf[p, b, r, pl.ds(off, 16)] = (
                                xbuf[p, b, r, pl.ds(off, 16)] + e)
                    return c2

                lax.fori_loop(0, NVJ, _add, 0)

            for cp in store_copies(k, p):
                cp.start()

            @pl.when(kk + 1 < NPAIR)
            def _prefetch():
                for cp in load_copies(k + 2, p):
                    cp.start()
        return carry

    lax.fori_loop(0, NPAIR, pair, 0)

    for p in range(2):
        for cp in store_copies(NCHUNK - 2 + p, p):
            cp.wait()


@functools.partial(
    pl.kernel,
    mesh=plsc.VectorSubcoreMesh(core_axis_name="c", subcore_axis_name="s"),
    out_type=jax.ShapeDtypeStruct((B, C, D), jnp.float32),
    scratch_types=[
        pltpu.VMEM((2, RC, D), jnp.float32),
        pltpu.VMEM((2, B, RC, D), jnp.float32),
        pltpu.VMEM((2, B, RC, D), jnp.float32),
        pltpu.SemaphoreType.DMA,
        pltpu.SemaphoreType.DMA,
        pltpu.SemaphoreType.DMA,
        pltpu.SemaphoreType.DMA,
    ],
)
def _sc_kernel(x_hbm, emb_hbm, out_hbm, ebuf, xbuf, obuf, sem_in0, sem_in1,
               sem_out0, sem_out1):
    _sc_body(x_hbm, emb_hbm, out_hbm, ebuf, xbuf, obuf, sem_in0, sem_in1,
             sem_out0, sem_out1)


def kernel(x, embedding):
    return _sc_kernel(x, embedding)
